# baseline (speedup 1.0000x reference)
import functools

import jax
import jax.numpy as jnp
from jax import lax
from jax.experimental import pallas as pl
from jax.experimental.pallas import tpu as pltpu
from jax.experimental.pallas import tpu_sc as plsc

_B = 16384
_DS = (32, 32, 16, 16)
_DTOT = 96


def _build():
    info = plsc.get_sparse_core_info()
    nc, ns = info.num_cores, info.num_subcores
    nw = nc * ns
    bpw = _B // nw

    mesh = plsc.VectorSubcoreMesh(core_axis_name="c", subcore_axis_name="s")

    @functools.partial(
        pl.kernel,
        mesh=mesh,
        out_type=jax.ShapeDtypeStruct((_B, _DTOT), jnp.float32),
        compiler_params=pltpu.CompilerParams(use_tc_tiling_on_sc=False),
        scratch_types=[
            pltpu.VMEM((4, bpw), jnp.int32),
            pltpu.VMEM((bpw, _DS[0]), jnp.float32),
            pltpu.VMEM((bpw, _DS[1]), jnp.float32),
            pltpu.VMEM((bpw, _DS[2]), jnp.float32),
            pltpu.VMEM((bpw, _DS[3]), jnp.float32),
            pltpu.SemaphoreType.DMA,
            pltpu.SemaphoreType.DMA,
            pltpu.SemaphoreType.DMA,
            pltpu.SemaphoreType.DMA,
        ],
    )
    def emb_kernel(u_hbm, i_hbm, c_hbm, b_hbm, wu, wi, wc, wb,
                   out_hbm, idx_v, r0, r1, r2, r3, s0, s1, s2, s3):
        wid = lax.axis_index("s") * nc + lax.axis_index("c")
        base = wid * bpw
        idx_refs = (u_hbm, i_hbm, c_hbm, b_hbm)
        tables = (wu, wi, wc, wb)
        rows = (r0, r1, r2, r3)
        sems = (s0, s1, s2, s3)
        copies = []
        for t in range(4):
            pltpu.sync_copy(idx_refs[t].at[pl.ds(base, bpw)], idx_v.at[t])
            copies.append(
                pltpu.async_copy(tables[t].at[idx_v.at[t]], rows[t], sems[t])
            )
        col = 0
        for t in range(4):
            copies[t].wait()
            pltpu.sync_copy(
                rows[t], out_hbm.at[pl.ds(base, bpw), pl.ds(col, _DS[t])]
            )
            col += _DS[t]

    return emb_kernel


_emb_kernel = _build()


def kernel(user_id, item_id, category, brand,
           W_user_id, W_item_id, W_category, W_brand):
    return _emb_kernel(user_id, item_id, category, brand,
                       W_user_id, W_item_id, W_category, W_brand)


# speedup vs baseline: 1.2208x; 1.2208x over previous
import functools

import jax
import jax.numpy as jnp
from jax import lax
from jax.experimental import pallas as pl
from jax.experimental.pallas import tpu as pltpu
from jax.experimental.pallas import tpu_sc as plsc

_B = 16384
_DS = (32, 32, 16, 16)
_COLS = (0, 32, 64, 80)
_DTOT = 96
_CH = 16


def _build():
    info = plsc.get_sparse_core_info()
    nc, ns = info.num_cores, info.num_subcores
    nw = nc * ns
    bpw = _B // nw

    mesh = plsc.VectorSubcoreMesh(core_axis_name="c", subcore_axis_name="s")

    @functools.partial(
        pl.kernel,
        mesh=mesh,
        out_type=jax.ShapeDtypeStruct((_B * _DTOT,), jnp.float32),
        scratch_types=[
            pltpu.VMEM((4, bpw), jnp.int32),
            pltpu.VMEM((_CH, 8, _DS[0]), jnp.float32),
            pltpu.VMEM((_CH, 8, _DS[1]), jnp.float32),
            pltpu.VMEM((_CH, 8, _DS[2]), jnp.float32),
            pltpu.VMEM((_CH, 8, _DS[3]), jnp.float32),
            pltpu.VMEM((bpw * _DTOT,), jnp.float32),
            pltpu.SemaphoreType.DMA,
        ],
    )
    def emb_kernel(u_hbm, i_hbm, c_hbm, b_hbm, wu, wi, wc, wb,
                   out_hbm, idx_v, sl0, sl1, sl2, sl3, rows_v, sem):
        wid = lax.axis_index("s") * nc + lax.axis_index("c")
        base = wid * bpw
        idx_refs = (u_hbm, i_hbm, c_hbm, b_hbm)
        tables = (wu, wi, wc, wb)
        slots = (sl0, sl1, sl2, sl3)
        for t in range(4):
            pltpu.sync_copy(idx_refs[t].at[pl.ds(base, bpw)], idx_v.at[t])

        def chunk(i, carry):
            vs = [idx_v[t, pl.ds(i * _CH, _CH)] for t in range(4)]
            cps = []
            for t in range(4):
                for j in range(_CH):
                    r = vs[t][j]
                    t_off = pl.multiple_of((r // 8) * 8, 8)
                    cps.append(
                        pltpu.async_copy(
                            tables[t].at[pl.ds(t_off, 8)], slots[t].at[j], sem
                        )
                    )
            for cp in cps:
                cp.wait()
            for t in range(4):
                for j in range(_CH):
                    r = vs[t][j]
                    row = lax.rem(r, 8)
                    for c in range(0, _DS[t], 16):
                        rows_v[
                            pl.ds((i * _CH + j) * _DTOT + _COLS[t] + c, 16)
                        ] = slots[t][j, row, pl.ds(c, 16)]
            return carry

        lax.fori_loop(0, bpw // _CH, chunk, 0)
        pltpu.sync_copy(rows_v, out_hbm.at[pl.ds(base * _DTOT, bpw * _DTOT)])

    return emb_kernel


_emb_kernel = _build()


def kernel(user_id, item_id, category, brand,
           W_user_id, W_item_id, W_category, W_brand):
    out = _emb_kernel(user_id, item_id, category, brand,
                      W_user_id, W_item_id, W_category, W_brand)
    return out.reshape(_B, _DTOT)
